# baseline (speedup 1.0000x reference)
import functools

import jax
import jax.numpy as jnp
from jax import lax
from jax.experimental import pallas as pl
from jax.experimental.pallas import tpu as pltpu
from jax.experimental.pallas import tpu_sc as plsc

_LANES = 16


def _make_sc_partials(batch, feat_dim, num_classes):
  info = plsc.get_sparse_core_info()
  nc, ns = info.num_cores, info.num_subcores
  nw = nc * ns
  assert batch % (8 * nw) == 0
  bpw = batch // nw
  ch = 32
  assert bpw % ch == 0
  groups = feat_dim // _LANES

  mesh = plsc.VectorSubcoreMesh(core_axis_name="c", subcore_axis_name="s")

  @functools.partial(
      pl.kernel,
      mesh=mesh,
      out_type=jax.ShapeDtypeStruct((nw, _LANES), jnp.float32),
      scratch_types=[
          pltpu.VMEM((ch,), jnp.int32),
          pltpu.VMEM((ch, feat_dim), jnp.float32),
          pltpu.VMEM((ch * feat_dim,), jnp.float32),
          pltpu.VMEM((_LANES,), jnp.float32),
          pltpu.SemaphoreType.DMA,
      ],
  )
  def sc_kernel(feat_hbm, lab_hbm, cent_hbm, out_hbm,
                idx_v, crows_v, f_v, acc_v, sem):
    wid = lax.axis_index("s") * nc + lax.axis_index("c")
    base = wid * bpw

    def sub_chunk(s, acc):
      row0 = base + s * ch
      pltpu.sync_copy(lab_hbm.at[pl.ds(row0, ch)], idx_v)
      gather = pltpu.async_copy(cent_hbm.at[idx_v], crows_v, sem)
      pltpu.sync_copy(feat_hbm.at[pl.ds(row0 * feat_dim, ch * feat_dim)], f_v)
      gather.wait()

      def row_body(r, a):
        for j in range(groups):
          f = f_v[pl.ds(r * feat_dim + j * _LANES, _LANES)]
          c = crows_v[r, pl.ds(j * _LANES, _LANES)]
          d = f - c
          a = a + d * d
        return a

      return lax.fori_loop(0, ch, row_body, acc)

    acc = lax.fori_loop(0, bpw // ch, sub_chunk,
                        jnp.zeros((_LANES,), jnp.float32))
    acc_v[...] = acc
    pltpu.sync_copy(acc_v, out_hbm.at[wid])

  return sc_kernel, nw


def _tc_reduce(partials, batch):
  def body(p_ref, o_ref):
    o_ref[0, 0] = jnp.sum(p_ref[...]) * (0.5 / batch)

  out = pl.pallas_call(
      body,
      out_shape=jax.ShapeDtypeStruct((1, 1), jnp.float32),
      out_specs=pl.BlockSpec(memory_space=pltpu.SMEM),
  )(partials)
  return out[0, 0]


def kernel(features, labels, centers):
  batch, feat_dim = features.shape
  num_classes = centers.shape[0]
  sc_kernel, nw = _make_sc_partials(batch, feat_dim, num_classes)
  partials = sc_kernel(
      features.reshape(batch * feat_dim),
      labels.astype(jnp.int32),
      centers,
  )
  return _tc_reduce(partials, batch)


# speedup vs baseline: 1.4102x; 1.4102x over previous
import functools

import jax
import jax.numpy as jnp
from jax import lax
from jax.experimental import pallas as pl
from jax.experimental.pallas import tpu as pltpu
from jax.experimental.pallas import tpu_sc as plsc

_LANES = 16


def _make_sc_partials(batch, feat_dim, num_classes):
  info = plsc.get_sparse_core_info()
  nc, ns = info.num_cores, info.num_subcores
  nw = nc * ns
  assert batch % (8 * nw) == 0
  bpw = batch // nw
  ch = 32
  assert bpw % ch == 0
  nsub = bpw // ch
  groups = feat_dim // _LANES

  mesh = plsc.VectorSubcoreMesh(core_axis_name="c", subcore_axis_name="s")

  @functools.partial(
      pl.kernel,
      mesh=mesh,
      out_type=jax.ShapeDtypeStruct((nw, _LANES), jnp.float32),
      scratch_types=[
          pltpu.VMEM((bpw,), jnp.int32),
          pltpu.VMEM((ch, feat_dim), jnp.float32),
          pltpu.VMEM((ch, feat_dim), jnp.float32),
          pltpu.VMEM((ch, feat_dim), jnp.float32),
          pltpu.VMEM((ch, feat_dim), jnp.float32),
          pltpu.VMEM((_LANES,), jnp.float32),
          pltpu.SemaphoreType.DMA,
          pltpu.SemaphoreType.DMA,
          pltpu.SemaphoreType.DMA,
          pltpu.SemaphoreType.DMA,
      ],
  )
  def sc_kernel(feat_hbm, lab_hbm, cent_hbm, out_hbm,
                idx_v, crows0, crows1, fb0, fb1, acc_v,
                gsem0, gsem1, fsem0, fsem1):
    wid = lax.axis_index("s") * nc + lax.axis_index("c")
    base = wid * bpw
    crows = (crows0, crows1)
    fbufs = (fb0, fb1)
    gsems = (gsem0, gsem1)
    fsems = (fsem0, fsem1)

    pltpu.sync_copy(lab_hbm.at[pl.ds(base, bpw)], idx_v)

    def issue(s):
      b = s % 2
      g = pltpu.async_copy(
          cent_hbm.at[idx_v.at[pl.ds(s * ch, ch)]], crows[b], gsems[b])
      f = pltpu.async_copy(
          feat_hbm.at[pl.ds(base + s * ch, ch)], fbufs[b], fsems[b])
      return g, f

    def accumulate(s, acc):
      b = s % 2
      fb, cb = fbufs[b], crows[b]

      def row_body(r, a):
        for j in range(groups):
          f = fb[r, pl.ds(j * _LANES, _LANES)]
          c = cb[r, pl.ds(j * _LANES, _LANES)]
          d = f - c
          a = a + d * d
        return a

      return lax.fori_loop(0, ch, row_body, acc)

    acc = jnp.zeros((_LANES,), jnp.float32)
    pending = issue(0)
    for s in range(nsub):
      nxt = issue(s + 1) if s + 1 < nsub else None
      pending[0].wait()
      pending[1].wait()
      acc = accumulate(s, acc)
      pending = nxt

    acc_v[...] = acc
    pltpu.sync_copy(acc_v, out_hbm.at[wid])

  return sc_kernel, nw


def _tc_reduce(partials, batch):
  def body(p_ref, o_ref):
    o_ref[0, 0] = jnp.sum(p_ref[...]) * (0.5 / batch)

  out = pl.pallas_call(
      body,
      out_shape=jax.ShapeDtypeStruct((1, 1), jnp.float32),
      out_specs=pl.BlockSpec(memory_space=pltpu.SMEM),
  )(partials)
  return out[0, 0]


def kernel(features, labels, centers):
  batch, feat_dim = features.shape
  num_classes = centers.shape[0]
  sc_kernel, nw = _make_sc_partials(batch, feat_dim, num_classes)
  partials = sc_kernel(features, labels.astype(jnp.int32), centers)
  return _tc_reduce(partials, batch)


# speedup vs baseline: 1.4642x; 1.0383x over previous
import functools

import jax
import jax.numpy as jnp
from jax import lax
from jax.experimental import pallas as pl
from jax.experimental.pallas import tpu as pltpu
from jax.experimental.pallas import tpu_sc as plsc

_LANES = 16


def _make_sc_partials(batch, feat_dim, num_classes):
  info = plsc.get_sparse_core_info()
  nc, ns = info.num_cores, info.num_subcores
  nw = nc * ns
  assert batch % (8 * nw) == 0
  bpw = batch // nw
  ch = 32
  assert bpw % ch == 0
  nsub = bpw // ch
  groups = feat_dim // _LANES

  mesh = plsc.VectorSubcoreMesh(core_axis_name="c", subcore_axis_name="s")

  @functools.partial(
      pl.kernel,
      mesh=mesh,
      out_type=jax.ShapeDtypeStruct((nw, _LANES), jnp.float32),
      scratch_types=[
          pltpu.VMEM((bpw,), jnp.int32),
          pltpu.VMEM((ch, feat_dim), jnp.float32),
          pltpu.VMEM((ch, feat_dim), jnp.float32),
          pltpu.VMEM((ch, feat_dim), jnp.float32),
          pltpu.VMEM((ch, feat_dim), jnp.float32),
          pltpu.VMEM((_LANES,), jnp.float32),
          pltpu.SemaphoreType.DMA,
          pltpu.SemaphoreType.DMA,
          pltpu.SemaphoreType.DMA,
          pltpu.SemaphoreType.DMA,
      ],
  )
  def sc_kernel(feat_hbm, lab_hbm, cent_hbm, out_hbm,
                idx_v, crows0, crows1, fb0, fb1, acc_v,
                gsem0, gsem1, fsem0, fsem1):
    wid = lax.axis_index("s") * nc + lax.axis_index("c")
    base = wid * bpw
    crows = (crows0, crows1)
    fbufs = (fb0, fb1)
    gsems = (gsem0, gsem1)
    fsems = (fsem0, fsem1)

    pltpu.sync_copy(lab_hbm.at[pl.ds(base, bpw)], idx_v)

    def issue(s):
      b = s % 2
      g = pltpu.async_copy(
          cent_hbm.at[idx_v.at[pl.ds(s * ch, ch)]], crows[b], gsems[b])
      f = pltpu.async_copy(
          feat_hbm.at[pl.ds(base + s * ch, ch)], fbufs[b], fsems[b])
      return g, f

    def accumulate(s, acc):
      b = s % 2
      fb, cb = fbufs[b], crows[b]

      unroll = 8
      jblocks = groups // unroll

      def blk_body(k, a):
        r = k // jblocks
        j0 = (k % jblocks) * unroll
        for j in range(unroll):
          f = fb[r, pl.ds((j0 + j) * _LANES, _LANES)]
          c = cb[r, pl.ds((j0 + j) * _LANES, _LANES)]
          d = f - c
          a = a + d * d
        return a

      return lax.fori_loop(0, ch * jblocks, blk_body, acc)

    acc = jnp.zeros((_LANES,), jnp.float32)
    pending = issue(0)
    for s in range(nsub):
      nxt = issue(s + 1) if s + 1 < nsub else None
      pending[0].wait()
      pending[1].wait()
      acc = accumulate(s, acc)
      pending = nxt

    acc_v[...] = acc
    pltpu.sync_copy(acc_v, out_hbm.at[wid])

  return sc_kernel, nw


def _tc_reduce(partials, batch):
  def body(p_ref, o_ref):
    o_ref[0, 0] = jnp.sum(p_ref[...]) * (0.5 / batch)

  out = pl.pallas_call(
      body,
      out_shape=jax.ShapeDtypeStruct((1, 1), jnp.float32),
      out_specs=pl.BlockSpec(memory_space=pltpu.SMEM),
  )(partials)
  return out[0, 0]


def kernel(features, labels, centers):
  batch, feat_dim = features.shape
  num_classes = centers.shape[0]
  sc_kernel, nw = _make_sc_partials(batch, feat_dim, num_classes)
  partials = sc_kernel(features, labels.astype(jnp.int32), centers)
  return _tc_reduce(partials, batch)
